# Initial kernel scaffold; baseline (speedup 1.0000x reference)
import jax
import jax.numpy as jnp
from jax.experimental import pallas as pl


def kernel(pos, table):
    raise NotImplementedError("write your pallas kernel here")


# speedup vs baseline: 2.1856x; 2.1856x over previous
import functools

import jax
import jax.numpy as jnp
from jax import lax
from jax.experimental import pallas as pl
from jax.experimental.pallas import tpu as pltpu
from jax.experimental.pallas import tpu_sc as plsc

NUM_CORES = 2
NUM_SUBCORES = 16
NUM_WORKERS = NUM_CORES * NUM_SUBCORES

CHUNK = 64


def _make_gather(B: int, V: int, D: int):
    b_per_w = B // NUM_WORKERS
    n_chunks = b_per_w // CHUNK
    mesh = plsc.VectorSubcoreMesh(core_axis_name="c", subcore_axis_name="s")

    @functools.partial(
        pl.kernel,
        mesh=mesh,
        out_type=jax.ShapeDtypeStruct((B, D), jnp.float32),
        scratch_types=[
            pltpu.VMEM((b_per_w,), jnp.int32),
            pltpu.VMEM((CHUNK, D), jnp.float32),
            pltpu.SemaphoreType.DMA,
        ],
    )
    def gather_kernel(pos_hbm, table_hbm, out_hbm, idx_v, rows_v, sem):
        wid = lax.axis_index("s") * NUM_CORES + lax.axis_index("c")
        base = wid * b_per_w
        pltpu.sync_copy(pos_hbm.at[pl.ds(base, b_per_w)], idx_v)

        def body(j, carry):
            off = j * CHUNK
            pltpu.async_copy(
                table_hbm.at[idx_v.at[pl.ds(off, CHUNK)]], rows_v, sem
            ).wait()
            pltpu.sync_copy(rows_v, out_hbm.at[pl.ds(base + off, CHUNK)])
            return carry

        lax.fori_loop(0, n_chunks, body, 0)

    return gather_kernel


def kernel(pos, table):
    V, D = table.shape
    flat_pos = pos.reshape(-1).astype(jnp.int32)
    B = flat_pos.shape[0]
    out = _make_gather(B, V, D)(flat_pos, table)
    return out.reshape(pos.shape + (D,))


# speedup vs baseline: 2.3677x; 1.0834x over previous
import functools

import jax
import jax.numpy as jnp
from jax import lax
from jax.experimental import pallas as pl
from jax.experimental.pallas import tpu as pltpu
from jax.experimental.pallas import tpu_sc as plsc

NUM_CORES = 2
NUM_SUBCORES = 16
NUM_WORKERS = NUM_CORES * NUM_SUBCORES

CHUNK = 32
NBUF = 2


def _make_gather(B: int, V: int, D: int):
    b_per_w = B // NUM_WORKERS
    n_chunks = b_per_w // CHUNK
    mesh = plsc.VectorSubcoreMesh(core_axis_name="c", subcore_axis_name="s")

    @functools.partial(
        pl.kernel,
        mesh=mesh,
        out_type=jax.ShapeDtypeStruct((B, D), jnp.float32),
        scratch_types=[
            pltpu.VMEM((b_per_w,), jnp.int32),
            pltpu.VMEM((CHUNK, D), jnp.float32),
            pltpu.VMEM((CHUNK, D), jnp.float32),
            pltpu.SemaphoreType.DMA,
            pltpu.SemaphoreType.DMA,
            pltpu.SemaphoreType.DMA,
            pltpu.SemaphoreType.DMA,
        ],
    )
    def gather_kernel(pos_hbm, table_hbm, out_hbm, idx_v, buf0, buf1,
                      g0, g1, w0, w1):
        wid = lax.axis_index("s") * NUM_CORES + lax.axis_index("c")
        base = wid * b_per_w
        bufs = (buf0, buf1)
        gsems = (g0, g1)
        wsems = (w0, w1)

        pltpu.sync_copy(pos_hbm.at[pl.ds(base, b_per_w)], idx_v)

        def gather_desc(j, b):
            return pltpu.make_async_copy(
                table_hbm.at[idx_v.at[pl.ds(j * CHUNK, CHUNK)]],
                bufs[b], gsems[b])

        def wb_desc(j, b):
            return pltpu.make_async_copy(
                bufs[b], out_hbm.at[pl.ds(base + j * CHUNK, CHUNK)],
                wsems[b])

        for b in range(NBUF):
            gather_desc(b, b).start()

        def body(k, carry):
            for b in range(NBUF):
                j = k * NBUF + b
                gather_desc(j, b).wait()
                wb_desc(j, b).start()
                wb_desc(j, b).wait()
                gather_desc(j + NBUF, b).start()
            return carry

        lax.fori_loop(0, n_chunks // NBUF - 1, body, 0)

        for b in range(NBUF):
            j = n_chunks - NBUF + b
            gather_desc(j, b).wait()
            wb_desc(j, b).start()
        for b in range(NBUF):
            j = n_chunks - NBUF + b
            wb_desc(j, b).wait()

    return gather_kernel


def kernel(pos, table):
    V, D = table.shape
    flat_pos = pos.reshape(-1).astype(jnp.int32)
    B = flat_pos.shape[0]
    out = _make_gather(B, V, D)(flat_pos, table)
    return out.reshape(pos.shape + (D,))


# speedup vs baseline: 2.3873x; 1.0083x over previous
import functools

import jax
import jax.numpy as jnp
from jax import lax
from jax.experimental import pallas as pl
from jax.experimental.pallas import tpu as pltpu
from jax.experimental.pallas import tpu_sc as plsc

NUM_CORES = 2
NUM_SUBCORES = 16
NUM_WORKERS = NUM_CORES * NUM_SUBCORES

CHUNK = 32
NBUF = 3


def _make_gather(B: int, V: int, D: int):
    b_per_w = B // NUM_WORKERS
    n_chunks = b_per_w // CHUNK
    mesh = plsc.VectorSubcoreMesh(core_axis_name="c", subcore_axis_name="s")

    @functools.partial(
        pl.kernel,
        mesh=mesh,
        out_type=jax.ShapeDtypeStruct((B, D), jnp.float32),
        scratch_types=(
            [pltpu.VMEM((b_per_w,), jnp.int32)]
            + [pltpu.VMEM((CHUNK, D), jnp.float32)] * NBUF
            + [pltpu.SemaphoreType.DMA] * (2 * NBUF)
        ),
    )
    def gather_kernel(pos_hbm, table_hbm, out_hbm, idx_v, *rest):
        bufs = rest[:NBUF]
        gsems = rest[NBUF:2 * NBUF]
        wsems = rest[2 * NBUF:]
        wid = lax.axis_index("s") * NUM_CORES + lax.axis_index("c")
        base = wid * b_per_w

        pltpu.sync_copy(pos_hbm.at[pl.ds(base, b_per_w)], idx_v)

        def gather_desc(j, b):
            return pltpu.make_async_copy(
                table_hbm.at[idx_v.at[pl.ds(j * CHUNK, CHUNK)]],
                bufs[b], gsems[b])

        def wb_desc(j, b):
            return pltpu.make_async_copy(
                bufs[b], out_hbm.at[pl.ds(base + j * CHUNK, CHUNK)],
                wsems[b])

        for b in range(NBUF):
            gather_desc(b, b).start()

        def step(j, b):
            gather_desc(j, b).wait()
            wb_desc(j, b).start()
            wb_desc(j, b).wait()
            gather_desc(j + NBUF, b).start()

        main = n_chunks - NBUF
        unrolled = (main // NBUF) * NBUF

        def body(k, carry):
            for b in range(NBUF):
                step(k * NBUF + b, b)
            return carry

        lax.fori_loop(0, main // NBUF, body, 0)

        for j in range(unrolled, main):
            step(j, j % NBUF)

        for j in range(main, n_chunks):
            gather_desc(j, j % NBUF).wait()
            wb_desc(j, j % NBUF).start()
        for j in range(main, n_chunks):
            wb_desc(j, j % NBUF).wait()

    return gather_kernel


def kernel(pos, table):
    V, D = table.shape
    flat_pos = pos.reshape(-1).astype(jnp.int32)
    B = flat_pos.shape[0]
    out = _make_gather(B, V, D)(flat_pos, table)
    return out.reshape(pos.shape + (D,))
